# Initial kernel scaffold; baseline (speedup 1.0000x reference)
import jax
import jax.numpy as jnp
from jax.experimental import pallas as pl


def kernel(x, edge_index, batch, W_l, b_l, W_r, b_r, att):
    raise NotImplementedError("write your pallas kernel here")


# speedup vs baseline: 44.7168x; 44.7168x over previous
import functools

import jax
import jax.numpy as jnp
from jax.experimental import pallas as pl
from jax.experimental.pallas import tpu as pltpu

NN = 50
TK = 20
EMB = 32


def _attn_kernel(x_ref, wl_ref, bl_ref, wr_ref, br_ref, att_ref,
                 val_ref, idx_ref, *, G):
    x = x_ref[...]
    xl = jnp.dot(x, wl_ref[...], preferred_element_type=jnp.float32) + bl_ref[...]
    xr = jnp.dot(x, wr_ref[...], preferred_element_type=jnp.float32) + br_ref[...]
    xl3 = xl.reshape(G, NN, EMB)
    xr3 = xr.reshape(G, NN, EMB)
    xl4 = jax.lax.broadcast_in_dim(xl3, (G, NN, NN, EMB), (0, 1, 3))
    xr4 = jax.lax.broadcast_in_dim(xr3, (G, NN, NN, EMB), (0, 2, 3))
    t = xl4 + xr4
    e = jnp.where(t >= 0, t, 0.2 * t)
    att4 = att_ref[...].reshape(1, 1, 1, EMB)
    s = jnp.sum(e * att4, axis=-1)
    m = jnp.max(s, axis=-1, keepdims=True)
    ex = jnp.exp(s - m)
    denom = jnp.sum(ex, axis=-1, keepdims=True)
    p = ex / (denom + 1e-16)
    ii = jax.lax.broadcasted_iota(jnp.int32, (G, NN, NN), 1)
    jj = jax.lax.broadcasted_iota(jnp.int32, (G, NN, NN), 2)
    p = jnp.where(ii == jj, -1.0, p)
    vals = []
    idxs = []
    for _ in range(TK):
        mv = jnp.max(p, axis=-1)
        am = jnp.argmax(p, axis=-1).astype(jnp.int32)
        vals.append(mv)
        idxs.append(am)
        p = jnp.where(jj == am[..., None], -2.0, p)
    val = jnp.stack(vals, axis=-1)
    idx = jnp.stack(idxs, axis=-1)
    base = (pl.program_id(0) * G + jax.lax.broadcasted_iota(
        jnp.int32, (G, 1, 1), 0)) * NN
    val_ref[...] = val
    idx_ref[...] = idx + base


def kernel(x, edge_index, batch, W_l, b_l, W_r, b_r, att):
    n_total = x.shape[0]
    b = n_total // NN
    G = 8
    grid = b // G
    val, idx = pl.pallas_call(
        functools.partial(_attn_kernel, G=G),
        grid=(grid,),
        in_specs=[
            pl.BlockSpec((G * NN, x.shape[1]), lambda i: (i, 0)),
            pl.BlockSpec((x.shape[1], EMB), lambda i: (0, 0)),
            pl.BlockSpec((1, EMB), lambda i: (0, 0)),
            pl.BlockSpec((x.shape[1], EMB), lambda i: (0, 0)),
            pl.BlockSpec((1, EMB), lambda i: (0, 0)),
            pl.BlockSpec((1, EMB), lambda i: (0, 0)),
        ],
        out_specs=[
            pl.BlockSpec((G, NN, TK), lambda i: (i, 0, 0)),
            pl.BlockSpec((G, NN, TK), lambda i: (i, 0, 0)),
        ],
        out_shape=[
            jax.ShapeDtypeStruct((b, NN, TK), jnp.float32),
            jax.ShapeDtypeStruct((b, NN, TK), jnp.int32),
        ],
    )(x, W_l, b_l.reshape(1, EMB), W_r, b_r.reshape(1, EMB), att)
    attention = val.reshape(-1)
    index_j = idx.reshape(-1)
    offsets = jnp.arange(b, dtype=jnp.int32) * NN
    index_i = (offsets[:, None]
               + jnp.repeat(jnp.arange(NN, dtype=jnp.int32), TK)[None, :]
               ).reshape(-1)
    new_edge_index = jnp.stack([index_i, index_j])
    return (new_edge_index, attention)


# speedup vs baseline: 118.8470x; 2.6578x over previous
import functools

import jax
import jax.numpy as jnp
from jax.experimental import pallas as pl
from jax.experimental.pallas import tpu as pltpu

NN = 50
TK = 20
EMB = 32
G = 8
JP = 64


def _proj_kernel(x_ref, w_ref, bt_ref, xl_ref, xr_ref):
    xlr = jnp.dot(x_ref[...], w_ref[...],
                  preferred_element_type=jnp.float32)
    xl_ref[...] = xlr[:, :EMB]
    xr_ref[...] = xlr[:, EMB:] + bt_ref[...]


def _attn_kernel(xl_ref, xrf_ref, delta_ref, onehot_ref, m_ref,
                 val_ref, idx_ref):
    R = G * NN
    u = jnp.concatenate([xl_ref[...], onehot_ref[...]], axis=1)
    w2 = jnp.concatenate([delta_ref[...], xrf_ref[...]], axis=0)
    t = jnp.dot(u, w2, preferred_element_type=jnp.float32)
    e = jnp.where(t >= 0, t, 0.2 * t)
    s = jnp.dot(e, m_ref[...], preferred_element_type=jnp.float32)
    jj = jax.lax.broadcasted_iota(jnp.int32, (R, JP), 1)
    s = jnp.where(jj >= NN, -jnp.inf, s)
    mx = jnp.max(s, axis=-1, keepdims=True)
    ex = jnp.exp(s - mx)
    denom = jnp.sum(ex, axis=-1, keepdims=True)
    p = ex / (denom + 1e-16)
    ii = jax.lax.broadcasted_iota(jnp.int32, (R, JP), 0) % NN
    p = jnp.where(ii == jj, -1.0, p)
    vals = []
    idxs = []
    for _ in range(TK):
        mv = jnp.max(p, axis=-1)
        am = jnp.argmax(p, axis=-1).astype(jnp.int32)
        vals.append(mv)
        idxs.append(am)
        p = jnp.where(jj == am[:, None], -2.0, p)
    val = jnp.stack(vals, axis=-1)
    idx = jnp.stack(idxs, axis=-1)
    row = jax.lax.broadcasted_iota(jnp.int32, (R, 1), 0)
    base = pl.program_id(0) * R + (row // NN) * NN
    val_ref[...] = val
    idx_ref[...] = idx + base


def kernel(x, edge_index, batch, W_l, b_l, W_r, b_r, att):
    n_total, IN = x.shape
    b = n_total // NN
    grid = b // G
    R = G * NN
    wcat = jnp.concatenate([W_l, W_r], axis=1)
    bt = (b_l + b_r)[None, :]
    xl, xr = pl.pallas_call(
        _proj_kernel,
        grid=(grid,),
        in_specs=[
            pl.BlockSpec((R, IN), lambda i: (i, 0)),
            pl.BlockSpec((IN, 2 * EMB), lambda i: (0, 0)),
            pl.BlockSpec((1, EMB), lambda i: (0, 0)),
        ],
        out_specs=[
            pl.BlockSpec((R, EMB), lambda i: (i, 0)),
            pl.BlockSpec((R, EMB), lambda i: (i, 0)),
        ],
        out_shape=[
            jax.ShapeDtypeStruct((n_total, EMB), jnp.float32),
            jax.ShapeDtypeStruct((n_total, EMB), jnp.float32),
        ],
    )(x, wcat, bt)
    xr_flat = xr.reshape(b, NN * EMB)
    delta = jnp.tile(jnp.eye(EMB, dtype=jnp.float32), (1, NN))
    onehot = (jnp.arange(R)[:, None] // NN
              == jnp.arange(G)[None, :]).astype(jnp.float32)
    m = jnp.concatenate(
        [jnp.kron(jnp.eye(NN, dtype=jnp.float32), att.reshape(EMB, 1)),
         jnp.zeros((NN * EMB, JP - NN), jnp.float32)], axis=1)
    val, idx = pl.pallas_call(
        _attn_kernel,
        grid=(grid,),
        in_specs=[
            pl.BlockSpec((R, EMB), lambda i: (i, 0)),
            pl.BlockSpec((G, NN * EMB), lambda i: (i, 0)),
            pl.BlockSpec((EMB, NN * EMB), lambda i: (0, 0)),
            pl.BlockSpec((R, G), lambda i: (0, 0)),
            pl.BlockSpec((NN * EMB, JP), lambda i: (0, 0)),
        ],
        out_specs=[
            pl.BlockSpec((R, TK), lambda i: (i, 0)),
            pl.BlockSpec((R, TK), lambda i: (i, 0)),
        ],
        out_shape=[
            jax.ShapeDtypeStruct((n_total, TK), jnp.float32),
            jax.ShapeDtypeStruct((n_total, TK), jnp.int32),
        ],
    )(xl, xr_flat, delta, onehot, m)
    attention = val.reshape(-1)
    index_j = idx.reshape(-1)
    offsets = jnp.arange(b, dtype=jnp.int32) * NN
    index_i = (offsets[:, None]
               + jnp.repeat(jnp.arange(NN, dtype=jnp.int32), TK)[None, :]
               ).reshape(-1)
    new_edge_index = jnp.stack([index_i, index_j])
    return (new_edge_index, attention)
